# baseline (speedup 1.0000x reference)
import functools

import jax
import jax.numpy as jnp
from jax import lax
from jax.experimental import pallas as pl
from jax.experimental.pallas import tpu as pltpu
from jax.experimental.pallas import tpu_sc as plsc

B, F, V, D, DD = 16384, 26, 100000, 32, 13
NF = F + DD
NC, NS, L = 2, 16, 16
NW = NC * NS
BPW = B // NW
CB = 16
NCH = BPW // CB
NB = 8


def _sc_embed(tables, sidx_t, dense_pad, W):
    mesh = plsc.VectorSubcoreMesh(core_axis_name="c", subcore_axis_name="s")

    @functools.partial(
        pl.kernel,
        mesh=mesh,
        out_type=jax.ShapeDtypeStruct((B, NF, D), jnp.float32),
        scratch_types=[
            pltpu.VMEM((F, BPW), jnp.int32),
            pltpu.VMEM((2, F, CB, D), jnp.float32),
            pltpu.VMEM((2, CB, NF, D), jnp.float32),
            pltpu.VMEM((BPW, L), jnp.float32),
            pltpu.VMEM((DD, DD * D), jnp.float32),
            pltpu.SemaphoreType.DMA,
            pltpu.SemaphoreType.DMA,
            pltpu.SemaphoreType.DMA,
            pltpu.SemaphoreType.DMA,
        ],
        compiler_params=pltpu.CompilerParams(use_tc_tiling_on_sc=False),
    )
    def k(tbl_hbm, idx_hbm, den_hbm, w_hbm, out_hbm,
          sidx_v, gtmp, obuf, den_v, w_v,
          gsem0, gsem1, osem0, osem1):
        wid = lax.axis_index("s") * NC + lax.axis_index("c")
        bbase = wid * BPW
        gsems = (gsem0, gsem1)
        osems = (osem0, osem1)

        pltpu.sync_copy(idx_hbm.at[:, pl.ds(bbase, BPW)], sidx_v)
        pltpu.sync_copy(den_hbm.at[pl.ds(bbase, BPW), :], den_v)
        pltpu.sync_copy(w_hbm, w_v)

        def fire_gathers(c, s):
            for f in range(F):
                pltpu.async_copy(
                    tbl_hbm.at[f].at[sidx_v.at[f, pl.ds(c * CB, CB)]],
                    gtmp.at[s, f],
                    gsems[s])

        def drain_gathers(s):
            pltpu.make_async_copy(
                tbl_hbm.at[pl.ds(0, F), pl.ds(0, CB), :],
                gtmp.at[s], gsems[s]).wait()

        def assemble(c, s):
            def per_batch(ci, carry):
                for f in range(F):
                    obuf[s, ci, f, pl.ds(0, L)] = gtmp[s, f, ci, pl.ds(0, L)]
                    obuf[s, ci, f, pl.ds(L, L)] = gtmp[s, f, ci, pl.ds(L, L)]
                return carry
            lax.fori_loop(0, CB, per_batch, None)

            def blk(c0, carry):
                base = c * CB + c0 * NB
                dvecs = [den_v[base + ci, pl.ds(0, L)] for ci in range(NB)]
                d_sc = [[dvecs[ci][kk] for kk in range(DD)]
                        for ci in range(NB)]

                def row(r, carry2):
                    wlo = [w_v[kk, pl.ds(r * D, L)] for kk in range(DD)]
                    whi = [w_v[kk, pl.ds(r * D + L, L)] for kk in range(DD)]
                    for ci in range(NB):
                        acc0 = d_sc[ci][0] * wlo[0]
                        acc1 = d_sc[ci][0] * whi[0]
                        for kk in range(1, DD):
                            acc0 = acc0 + d_sc[ci][kk] * wlo[kk]
                            acc1 = acc1 + d_sc[ci][kk] * whi[kk]
                        obuf[s, c0 * NB + ci, F + r, pl.ds(0, L)] = acc0
                        obuf[s, c0 * NB + ci, F + r, pl.ds(L, L)] = acc1
                    return carry2
                lax.fori_loop(0, DD, row, None)
                return carry
            lax.fori_loop(0, CB // NB, blk, None)

        def fire_outcopy(c, s):
            pltpu.async_copy(
                obuf.at[s],
                out_hbm.at[pl.ds(bbase + c * CB, CB), :, :],
                osems[s])

        def drain_outcopy(s):
            pltpu.make_async_copy(
                out_hbm.at[pl.ds(0, CB), :, :], obuf.at[s],
                osems[s]).wait()

        fire_gathers(0, 0)

        def body(cc, carry):
            ca = 2 * cc
            cb = 2 * cc + 1
            drain_gathers(0)
            fire_gathers(cb, 1)

            @pl.when(cc > 0)
            def _():
                drain_outcopy(0)
            assemble(ca, 0)
            fire_outcopy(ca, 0)

            drain_gathers(1)

            @pl.when(cc < NCH // 2 - 1)
            def _():
                fire_gathers(ca + 2, 0)

            @pl.when(cc > 0)
            def _():
                drain_outcopy(1)
            assemble(cb, 1)
            fire_outcopy(cb, 1)
            return carry
        lax.fori_loop(0, NCH // 2, body, None)
        drain_outcopy(0)
        drain_outcopy(1)

    return k(tables, sidx_t, dense_pad, W)


def kernel(sparse_inputs, dense_inputs, tables, W):
    sidx_t = sparse_inputs.T.astype(jnp.int32)
    dense_pad = jnp.pad(dense_inputs, ((0, 0), (0, L - DD)))
    return _sc_embed(tables, sidx_t, dense_pad, W)


# speedup vs baseline: 1.6487x; 1.6487x over previous
import functools

import jax
import jax.numpy as jnp
from jax import lax
from jax.experimental import pallas as pl
from jax.experimental.pallas import tpu as pltpu
from jax.experimental.pallas import tpu_sc as plsc

B, F, V, D, DD = 16384, 26, 100000, 32, 13
NF = F + DD
NC, NS, L = 2, 16, 16
NW = NC * NS
BH = 8192
DSEG = 512
NSEG = B // DSEG
CPT = 416 // NW
CBL = 3


def _sc_embed_t(tables_t, sidx_t, den_t, w_rep):
    mesh = plsc.VectorSubcoreMesh(core_axis_name="c", subcore_axis_name="s")

    @functools.partial(
        pl.kernel,
        mesh=mesh,
        out_type=jax.ShapeDtypeStruct((NF, D, B), jnp.float32),
        scratch_types=[
            pltpu.VMEM((V,), jnp.float32),
            pltpu.VMEM((BH,), jnp.int32),
            pltpu.VMEM((BH,), jnp.float32),
            pltpu.VMEM((DD, DSEG), jnp.float32),
            pltpu.VMEM((CPT, DD, L), jnp.float32),
            pltpu.VMEM((CBL, DSEG), jnp.float32),
            pltpu.SemaphoreType.DMA,
            pltpu.SemaphoreType.DMA,
            pltpu.SemaphoreType.DMA,
        ],
        compiler_params=pltpu.CompilerParams(use_tc_tiling_on_sc=False,
                                             needs_layout_passes=False),
    )
    def k(tbl_hbm, idx_hbm, den_hbm, w_hbm, out_hbm,
          tbuf, ibuf, rbuf, dseg, wbuf, drbuf, tsem, osem, dsem):
        t = lax.axis_index("s") * NC + lax.axis_index("c")

        pltpu.sync_copy(w_hbm.at[pl.ds(t * CPT, CPT)], wbuf)

        def dense_seg(seg, carry):
            pltpu.sync_copy(den_hbm.at[:, pl.ds(seg * DSEG, DSEG)], dseg)
            for cb0 in range(0, CPT, CBL):
                ncb = min(CBL, CPT - cb0)
                wv = [[wbuf[cb0 + cc, kk, pl.ds(0, L)] for kk in range(DD)]
                      for cc in range(ncb)]

                def chunk(i, carry2):
                    dv = [dseg[kk, pl.ds(i * L, L)] for kk in range(DD)]
                    for cc in range(ncb):
                        acc = dv[0] * wv[cc][0]
                        for kk in range(1, DD):
                            acc = acc + dv[kk] * wv[cc][kk]
                        drbuf[cc, pl.ds(i * L, L)] = acc
                    return carry2
                lax.fori_loop(0, DSEG // L, chunk, None)

                for cc in range(ncb):
                    col = t * CPT + cb0 + cc
                    pltpu.async_copy(
                        drbuf.at[cc],
                        out_hbm.at[F + col // D, lax.rem(col, D),
                                   pl.ds(seg * DSEG, DSEG)],
                        dsem)
                for cc in range(ncb):
                    pltpu.make_async_copy(
                        out_hbm.at[0, 0, pl.ds(0, DSEG)], drbuf.at[cc],
                        dsem).wait()
            return carry
        lax.fori_loop(0, NSEG, dense_seg, None)

        pltpu.async_copy(tbl_hbm.at[0, t, :], tbuf, tsem)
        for f in range(F):
            pltpu.make_async_copy(tbl_hbm.at[0, t, :], tbuf, tsem).wait()
            for h in range(2):
                pltpu.sync_copy(idx_hbm.at[f, pl.ds(h * BH, BH)], ibuf)

                def g(i, carry):
                    iv = ibuf[pl.ds(i * L, L)]
                    rbuf[pl.ds(i * L, L)] = plsc.load_gather(tbuf, [iv])
                    return carry
                lax.fori_loop(0, BH // L, g, None)
                pltpu.async_copy(
                    rbuf, out_hbm.at[f, t, pl.ds(h * BH, BH)], osem)
                pltpu.make_async_copy(
                    out_hbm.at[0, 0, pl.ds(0, BH)], rbuf, osem).wait()
            if f + 1 < F:
                pltpu.async_copy(tbl_hbm.at[f + 1, t, :], tbuf, tsem)

    return k(tables_t, sidx_t, den_t, w_rep)


def kernel(sparse_inputs, dense_inputs, tables, W):
    tables_t = jnp.transpose(tables, (0, 2, 1))
    sidx_t = sparse_inputs.T.astype(jnp.int32)
    den_t = dense_inputs.T
    w_rep = jnp.broadcast_to(W.T[:, :, None], (DD * D, DD, L))
    out_t = _sc_embed_t(tables_t, sidx_t, den_t, w_rep)
    return jnp.transpose(out_t, (2, 0, 1))


# speedup vs baseline: 1.7086x; 1.0363x over previous
import functools

import jax
import jax.numpy as jnp
from jax import lax
from jax.experimental import pallas as pl
from jax.experimental.pallas import tpu as pltpu
from jax.experimental.pallas import tpu_sc as plsc

B, F, V, D, DD = 16384, 26, 100000, 32, 13
NF = F + DD
NC, NS, L = 2, 16, 16
NW = NC * NS
BH = 8192
DSEG = 512
NSEG = B // DSEG
CPT = 416 // NW
CBL = 3


def _sc_embed_t(tables_t, sidx_t, den_t, w_rep):
    mesh = plsc.VectorSubcoreMesh(core_axis_name="c", subcore_axis_name="s")

    @functools.partial(
        pl.kernel,
        mesh=mesh,
        out_type=jax.ShapeDtypeStruct((NF, D, B), jnp.float32),
        scratch_types=[
            pltpu.VMEM((V,), jnp.float32),
            pltpu.VMEM((BH,), jnp.int32),
            pltpu.VMEM((BH,), jnp.float32),
            pltpu.VMEM((DD, DSEG), jnp.float32),
            pltpu.VMEM((CPT, DD, L), jnp.float32),
            pltpu.VMEM((CBL, DSEG), jnp.float32),
            pltpu.SemaphoreType.DMA,
            pltpu.SemaphoreType.DMA,
            pltpu.SemaphoreType.DMA,
        ],
        compiler_params=pltpu.CompilerParams(use_tc_tiling_on_sc=False,
                                             needs_layout_passes=False),
    )
    def k(tbl_hbm, idx_hbm, den_hbm, w_hbm, out_hbm,
          tbuf, ibuf, rbuf, dseg, wbuf, drbuf, tsem, osem, dsem):
        t = lax.axis_index("s") * NC + lax.axis_index("c")

        pltpu.sync_copy(w_hbm.at[pl.ds(t * CPT, CPT)], wbuf)

        def dense_seg(seg, carry):
            pltpu.sync_copy(den_hbm.at[:, pl.ds(seg * DSEG, DSEG)], dseg)
            for cb0 in range(0, CPT, CBL):
                ncb = min(CBL, CPT - cb0)
                wv = [[wbuf[cb0 + cc, kk, pl.ds(0, L)] for kk in range(DD)]
                      for cc in range(ncb)]

                def chunk(i, carry2):
                    for u in range(2):
                        o = i * 2 * L + u * L
                        dv = [dseg[kk, pl.ds(o, L)] for kk in range(DD)]
                        for cc in range(ncb):
                            acc = dv[0] * wv[cc][0]
                            for kk in range(1, DD):
                                acc = acc + dv[kk] * wv[cc][kk]
                            drbuf[cc, pl.ds(o, L)] = acc
                    return carry2
                lax.fori_loop(0, DSEG // (2 * L), chunk, None)

                for cc in range(ncb):
                    col = t * CPT + cb0 + cc
                    pltpu.async_copy(
                        drbuf.at[cc],
                        out_hbm.at[F + col // D, lax.rem(col, D),
                                   pl.ds(seg * DSEG, DSEG)],
                        dsem)
                for cc in range(ncb):
                    pltpu.make_async_copy(
                        out_hbm.at[0, 0, pl.ds(0, DSEG)], drbuf.at[cc],
                        dsem).wait()
            return carry
        lax.fori_loop(0, NSEG, dense_seg, None)

        pltpu.async_copy(tbl_hbm.at[0, t, :], tbuf, tsem)
        for f in range(F):
            pltpu.make_async_copy(tbl_hbm.at[0, t, :], tbuf, tsem).wait()
            for h in range(2):
                pltpu.sync_copy(idx_hbm.at[f, pl.ds(h * BH, BH)], ibuf)

                def g(i, carry):
                    for u in range(8):
                        iv = ibuf[pl.ds(i * 8 * L + u * L, L)]
                        rbuf[pl.ds(i * 8 * L + u * L, L)] = (
                            plsc.load_gather(tbuf, [iv]))
                    return carry
                lax.fori_loop(0, BH // (8 * L), g, None)
                pltpu.async_copy(
                    rbuf, out_hbm.at[f, t, pl.ds(h * BH, BH)], osem)
                pltpu.make_async_copy(
                    out_hbm.at[0, 0, pl.ds(0, BH)], rbuf, osem).wait()
            if f + 1 < F:
                pltpu.async_copy(tbl_hbm.at[f + 1, t, :], tbuf, tsem)

    return k(tables_t, sidx_t, den_t, w_rep)


def kernel(sparse_inputs, dense_inputs, tables, W):
    tables_t = jnp.transpose(tables, (0, 2, 1))
    sidx_t = sparse_inputs.T.astype(jnp.int32)
    den_t = dense_inputs.T
    w_rep = jnp.broadcast_to(W.T[:, :, None], (DD * D, DD, L))
    out_t = _sc_embed_t(tables_t, sidx_t, den_t, w_rep)
    return jnp.transpose(out_t, (2, 0, 1))
